# Initial kernel scaffold; baseline (speedup 1.0000x reference)
import jax
import jax.numpy as jnp
from jax.experimental import pallas as pl


def kernel(x, noise, sqrt_alphas_cumprod, sqrt_one_minus_alphas_cumprod, t):
    raise NotImplementedError("write your pallas kernel here")


# speedup vs baseline: 4.3565x; 4.3565x over previous
import functools

import jax
import jax.numpy as jnp
from jax import lax
from jax.experimental import pallas as pl
from jax.experimental.pallas import tpu as pltpu
from jax.experimental.pallas import tpu_sc as plsc

_N = 16384
_D = 128
_TPAD = 1024
_TW = 16
_NW = 32
_CHUNK = _N // _NW


def _sc_gather(t, tab):
    mesh = plsc.VectorSubcoreMesh(core_axis_name="c", subcore_axis_name="s")

    @functools.partial(
        pl.kernel,
        mesh=mesh,
        out_type=jax.ShapeDtypeStruct((_N, _TW), jnp.float32),
        scratch_types=[
            pltpu.VMEM((_CHUNK,), jnp.int32),
            pltpu.VMEM((_CHUNK, _TW), jnp.float32),
            pltpu.SemaphoreType.DMA,
        ],
        compiler_params=pltpu.CompilerParams(use_tc_tiling_on_sc=False),
    )
    def gather_kernel(t_hbm, tab_hbm, g_hbm, idx_v, rows_v, sem):
        wid = lax.axis_index("s") * 2 + lax.axis_index("c")
        base = wid * _CHUNK
        pltpu.sync_copy(t_hbm.at[pl.ds(base, _CHUNK)], idx_v)
        pltpu.async_copy(tab_hbm.at[idx_v], rows_v, sem).wait()
        pltpu.sync_copy(rows_v, g_hbm.at[pl.ds(base, _CHUNK)])

    return gather_kernel(t, tab)


def _tc_fma(x, noise, g):
    rows = 2048
    grid = (_N // rows,)

    def body(x_ref, n_ref, g_ref, o_ref):
        c1 = g_ref[:, 0:1]
        c2 = g_ref[:, 1:2]
        o_ref[...] = c1 * x_ref[...] + c2 * n_ref[...]

    return pl.pallas_call(
        body,
        grid=grid,
        in_specs=[
            pl.BlockSpec((rows, _D), lambda i: (i, 0)),
            pl.BlockSpec((rows, _D), lambda i: (i, 0)),
            pl.BlockSpec((rows, _TW), lambda i: (i, 0)),
        ],
        out_specs=pl.BlockSpec((rows, _D), lambda i: (i, 0)),
        out_shape=jax.ShapeDtypeStruct((_N, _D), jnp.float32),
    )(x, noise, g)


def kernel(x, noise, sqrt_alphas_cumprod, sqrt_one_minus_alphas_cumprod, t):
    tab = jnp.zeros((_TPAD, _TW), dtype=jnp.float32)
    tab = tab.at[: sqrt_alphas_cumprod.shape[0], 0].set(sqrt_alphas_cumprod)
    tab = tab.at[: sqrt_one_minus_alphas_cumprod.shape[0], 1].set(
        sqrt_one_minus_alphas_cumprod)
    g = _sc_gather(t.astype(jnp.int32), tab)
    return _tc_fma(x, noise, g)


# speedup vs baseline: 5.3386x; 1.2254x over previous
import functools

import jax
import jax.numpy as jnp
from jax import lax
from jax.experimental import pallas as pl
from jax.experimental.pallas import tpu as pltpu
from jax.experimental.pallas import tpu_sc as plsc

_N = 16384
_D = 128
_TPAD = 1024
_LANES = 16
_NW = 32
_CHUNK = _N // _NW
_ROWS = 256
_NCH = _CHUNK // _ROWS


def _sc_qsample(x, noise, tab1, tab2, t):
    mesh = plsc.VectorSubcoreMesh(core_axis_name="c", subcore_axis_name="s")

    @functools.partial(
        pl.kernel,
        mesh=mesh,
        out_type=jax.ShapeDtypeStruct((_N, _D), jnp.float32),
        scratch_types=[
            pltpu.VMEM((_CHUNK,), jnp.int32),
            pltpu.VMEM((_CHUNK, _LANES), jnp.float32),
            pltpu.VMEM((_CHUNK, _LANES), jnp.float32),
            pltpu.VMEM((_ROWS, _D), jnp.float32),
            pltpu.VMEM((_ROWS, _D), jnp.float32),
            pltpu.VMEM((_ROWS, _D), jnp.float32),
            pltpu.SemaphoreType.DMA,
            pltpu.SemaphoreType.DMA,
        ],
        compiler_params=pltpu.CompilerParams(use_tc_tiling_on_sc=False),
    )
    def qsample_kernel(x_hbm, n_hbm, tab1_hbm, tab2_hbm, t_hbm, o_hbm,
                       idx_v, c1_v, c2_v, xv, nv, ov, sem1, sem2):
        wid = lax.axis_index("s") * 2 + lax.axis_index("c")
        base = wid * _CHUNK
        pltpu.sync_copy(t_hbm.at[pl.ds(base, _CHUNK)], idx_v)
        cp1 = pltpu.async_copy(tab1_hbm.at[idx_v], c1_v, sem1)
        cp2 = pltpu.async_copy(tab2_hbm.at[idx_v], c2_v, sem2)
        cp1.wait()
        cp2.wait()

        for ch in range(_NCH):
            cbase = base + ch * _ROWS
            cpx = pltpu.async_copy(x_hbm.at[pl.ds(cbase, _ROWS)], xv, sem1)
            cpn = pltpu.async_copy(n_hbm.at[pl.ds(cbase, _ROWS)], nv, sem2)
            cpx.wait()
            cpn.wait()

            def body(r, carry, ch=ch):
                c1 = c1_v[ch * _ROWS + r, :]
                c2 = c2_v[ch * _ROWS + r, :]
                for j in range(_D // _LANES):
                    sl = pl.ds(j * _LANES, _LANES)
                    ov[r, sl] = c1 * xv[r, sl] + c2 * nv[r, sl]
                return carry

            lax.fori_loop(0, _ROWS, body, 0)
            pltpu.sync_copy(ov, o_hbm.at[pl.ds(cbase, _ROWS)])

    return qsample_kernel(x, noise, tab1, tab2, t)


def kernel(x, noise, sqrt_alphas_cumprod, sqrt_one_minus_alphas_cumprod, t):
    pad1 = jnp.pad(sqrt_alphas_cumprod,
                   (0, _TPAD - sqrt_alphas_cumprod.shape[0]))
    pad2 = jnp.pad(sqrt_one_minus_alphas_cumprod,
                   (0, _TPAD - sqrt_one_minus_alphas_cumprod.shape[0]))
    tab1 = jnp.broadcast_to(pad1[:, None], (_TPAD, _LANES))
    tab2 = jnp.broadcast_to(pad2[:, None], (_TPAD, _LANES))
    return _sc_qsample(x, noise, tab1, tab2, t.astype(jnp.int32))


# speedup vs baseline: 5.9168x; 1.1083x over previous
import functools

import jax
import jax.numpy as jnp
from jax import lax
from jax.experimental import pallas as pl
from jax.experimental.pallas import tpu as pltpu
from jax.experimental.pallas import tpu_sc as plsc

_N = 16384
_D = 128
_TPAD = 1024
_LANES = 16
_NW = 32
_CHUNK = _N // _NW
_ROWS = 128
_NCH = _CHUNK // _ROWS


def _sc_qsample(x, noise, tab1, tab2, t):
    mesh = plsc.VectorSubcoreMesh(core_axis_name="c", subcore_axis_name="s")

    @functools.partial(
        pl.kernel,
        mesh=mesh,
        out_type=jax.ShapeDtypeStruct((_N, _D), jnp.float32),
        scratch_types=[
            pltpu.VMEM((_CHUNK,), jnp.int32),
            pltpu.VMEM((_CHUNK, _LANES), jnp.float32),
            pltpu.VMEM((_CHUNK, _LANES), jnp.float32),
            [pltpu.VMEM((_ROWS, _D), jnp.float32)] * 2,
            [pltpu.VMEM((_ROWS, _D), jnp.float32)] * 2,
            [pltpu.VMEM((_ROWS, _D), jnp.float32)] * 2,
            [pltpu.SemaphoreType.DMA] * 2,
            [pltpu.SemaphoreType.DMA] * 2,
            [pltpu.SemaphoreType.DMA] * 2,
        ],
        compiler_params=pltpu.CompilerParams(use_tc_tiling_on_sc=False),
    )
    def qsample_kernel(x_hbm, n_hbm, tab1_hbm, tab2_hbm, t_hbm, o_hbm,
                       idx_v, c1_v, c2_v, xbufs, nbufs, obufs,
                       sxs, sns, sos):
        wid = lax.axis_index("s") * 2 + lax.axis_index("c")
        base = wid * _CHUNK
        pltpu.sync_copy(t_hbm.at[pl.ds(base, _CHUNK)], idx_v)
        cp1 = pltpu.async_copy(tab1_hbm.at[idx_v], c1_v, sxs[0])
        cp2 = pltpu.async_copy(tab2_hbm.at[idx_v], c2_v, sns[0])
        cp1.wait()
        cp2.wait()

        def start_in(ch):
            b = ch % 2
            cx = pltpu.async_copy(
                x_hbm.at[pl.ds(base + ch * _ROWS, _ROWS)], xbufs[b], sxs[b])
            cn = pltpu.async_copy(
                n_hbm.at[pl.ds(base + ch * _ROWS, _ROWS)], nbufs[b], sns[b])
            return cx, cn

        in_flight = [start_in(0)]
        out_flight = [None, None]
        for ch in range(_NCH):
            b = ch % 2
            if ch + 1 < _NCH:
                in_flight.append(start_in(ch + 1))
            cx, cn = in_flight[ch]
            cx.wait()
            cn.wait()
            if out_flight[b] is not None:
                out_flight[b].wait()

            def body(r, carry, ch=ch, b=b):
                c1 = c1_v[ch * _ROWS + r, :]
                c2 = c2_v[ch * _ROWS + r, :]
                for j in range(_D // _LANES):
                    sl = pl.ds(j * _LANES, _LANES)
                    obufs[b][r, sl] = c1 * xbufs[b][r, sl] + c2 * nbufs[b][r, sl]
                return carry

            lax.fori_loop(0, _ROWS, body, 0)
            out_flight[b] = pltpu.async_copy(
                obufs[b], o_hbm.at[pl.ds(base + ch * _ROWS, _ROWS)], sos[b])
        for cp in out_flight:
            if cp is not None:
                cp.wait()

    return qsample_kernel(x, noise, tab1, tab2, t)


def kernel(x, noise, sqrt_alphas_cumprod, sqrt_one_minus_alphas_cumprod, t):
    pad1 = jnp.pad(sqrt_alphas_cumprod,
                   (0, _TPAD - sqrt_alphas_cumprod.shape[0]))
    pad2 = jnp.pad(sqrt_one_minus_alphas_cumprod,
                   (0, _TPAD - sqrt_one_minus_alphas_cumprod.shape[0]))
    tab1 = jnp.broadcast_to(pad1[:, None], (_TPAD, _LANES))
    tab2 = jnp.broadcast_to(pad2[:, None], (_TPAD, _LANES))
    return _sc_qsample(x, noise, tab1, tab2, t.astype(jnp.int32))


# speedup vs baseline: 5.9801x; 1.0107x over previous
import functools

import jax
import jax.numpy as jnp
from jax import lax
from jax.experimental import pallas as pl
from jax.experimental.pallas import tpu as pltpu
from jax.experimental.pallas import tpu_sc as plsc

_N = 16384
_D = 128
_T = 1000
_LANES = 16
_NW = 32
_CHUNK = _N // _NW
_ROWS = 128
_NCH = _CHUNK // _ROWS


def _sc_qsample(x, noise, tab1, tab2, t):
    mesh = plsc.VectorSubcoreMesh(core_axis_name="c", subcore_axis_name="s")

    @functools.partial(
        pl.kernel,
        mesh=mesh,
        out_type=jax.ShapeDtypeStruct((_N, _D), jnp.float32),
        scratch_types=[
            pltpu.VMEM((_CHUNK,), jnp.int32),
            pltpu.VMEM((_T + _LANES,), jnp.float32),
            pltpu.VMEM((_T + _LANES,), jnp.float32),
            [pltpu.VMEM((_ROWS, _D), jnp.float32)] * 2,
            [pltpu.VMEM((_ROWS, _D), jnp.float32)] * 2,
            [pltpu.VMEM((_ROWS, _D), jnp.float32)] * 2,
            [pltpu.SemaphoreType.DMA] * 2,
            [pltpu.SemaphoreType.DMA] * 2,
            [pltpu.SemaphoreType.DMA] * 2,
        ],
        compiler_params=pltpu.CompilerParams(use_tc_tiling_on_sc=False),
    )
    def qsample_kernel(x_hbm, n_hbm, tab1_hbm, tab2_hbm, t_hbm, o_hbm,
                       idx_v, t1_v, t2_v, xbufs, nbufs, obufs,
                       sxs, sns, sos):
        wid = lax.axis_index("s") * 2 + lax.axis_index("c")
        base = wid * _CHUNK

        def start_in(ch):
            b = ch % 2
            cx = pltpu.async_copy(
                x_hbm.at[pl.ds(base + ch * _ROWS, _ROWS)], xbufs[b], sxs[b])
            cn = pltpu.async_copy(
                n_hbm.at[pl.ds(base + ch * _ROWS, _ROWS)], nbufs[b], sns[b])
            return cx, cn

        in_flight = [start_in(0)]
        pltpu.sync_copy(t_hbm.at[pl.ds(base, _CHUNK)], idx_v)
        pltpu.sync_copy(tab1_hbm, t1_v.at[pl.ds(0, _T)])
        pltpu.sync_copy(tab2_hbm, t2_v.at[pl.ds(0, _T)])

        out_flight = [None, None]
        for ch in range(_NCH):
            b = ch % 2
            if ch + 1 < _NCH:
                in_flight.append(start_in(ch + 1))
            cx, cn = in_flight[ch]
            cx.wait()
            cn.wait()
            if out_flight[b] is not None:
                out_flight[b].wait()

            def body(g, carry, ch=ch, b=b):
                rbase = g * _LANES
                idxv = idx_v[pl.ds(ch * _ROWS + rbase, _LANES)]
                for i in range(_LANES):
                    ti = idxv[i]
                    c1 = t1_v[pl.ds(ti, _LANES)][0]
                    c2 = t2_v[pl.ds(ti, _LANES)][0]
                    r = rbase + i
                    for j in range(_D // _LANES):
                        sl = pl.ds(j * _LANES, _LANES)
                        obufs[b][r, sl] = (c1 * xbufs[b][r, sl]
                                           + c2 * nbufs[b][r, sl])
                return carry

            lax.fori_loop(0, _ROWS // _LANES, body, 0)
            out_flight[b] = pltpu.async_copy(
                obufs[b], o_hbm.at[pl.ds(base + ch * _ROWS, _ROWS)], sos[b])
        for cp in out_flight:
            if cp is not None:
                cp.wait()

    return qsample_kernel(x, noise, tab1, tab2, t)


def kernel(x, noise, sqrt_alphas_cumprod, sqrt_one_minus_alphas_cumprod, t):
    return _sc_qsample(x, noise, sqrt_alphas_cumprod,
                       sqrt_one_minus_alphas_cumprod, t.astype(jnp.int32))
